# baseline (speedup 1.0000x reference)
import functools

import jax
import jax.numpy as jnp
from jax import lax
from jax.experimental import pallas as pl
from jax.experimental.pallas import tpu as pltpu
from jax.experimental.pallas import tpu_sc as plsc

_B = 128
_D = 64
_N = 32768
_K = 32
_NT = 16
_TN = _N // _NT
_CH = 128
_NCH = _N // _CH
_CPT = _TN // _CH
_MAXC = 48
_CAND = 256
_NSC = 32
_RPS = _B // _NSC


def _scores_tile(x_ref, w_ref, b_ref, h_ref):
    xw = lax.dot_general(x_ref[:], w_ref[:], (((1,), (1,)), ((), ())),
                         preferred_element_type=jnp.float32)
    sig = 1.0 / (1.0 + jnp.exp(-h_ref[:]))
    return jnp.maximum(xw + b_ref[:], 0.0) * sig


def _score_kernel(x_ref, w_ref, b_ref, h_ref, s_out, t32_ref, cm_out, cm_ref):
    i = pl.program_id(0)
    s = _scores_tile(x_ref, w_ref, b_ref, h_ref)
    s_out[:] = s
    mx = jnp.max(s.reshape(_B, _CPT, _CH), axis=2)
    cm_ref[i] = mx
    cm_out[0] = mx

    @pl.when(i == _NT - 1)
    def _():
        def body(k, p):
            cand = p | (1 << (30 - k))
            cf = lax.bitcast_convert_type(cand, jnp.float32)
            cnt = jnp.sum((cm_ref[:] >= cf[None]).astype(jnp.int32),
                          axis=(0, 2))[:, None]
            return jnp.where(cnt >= _K, cand, p)

        t_int = lax.fori_loop(0, 31, body, jnp.zeros((_B, 1), jnp.int32))
        t32_ref[:] = jnp.broadcast_to(
            lax.bitcast_convert_type(t_int, jnp.float32), (_B, 128))


def _mask_kernel(x_ref, w_ref, b_ref, h_ref, tval_ref, tidx_ref, o_ref):
    i = pl.program_id(0)
    s = _scores_tile(x_ref, w_ref, b_ref, h_ref)
    t = tval_ref[:, 0:1]
    it = tidx_ref[:, 0:1]
    col = i * _TN + lax.broadcasted_iota(jnp.int32, (_B, _TN), 1)
    keep = (s > t) | ((s == t) & (col <= it))
    o_ref[:] = jnp.where(keep, s, 0.0)


def _iota16():
    return lax.broadcasted_iota(jnp.int32, (16,), 0)


def _splat(v, dt=jnp.int32):
    return jnp.full((16,), v, dt)


def _sc_select(sc2_ref, cmf_ref, t32f_ref, ctab_ref, tout_ref, iout_ref,
               cm_v, t32_v, gidx_v, lidx_v, rows_v, cols_v, vals_s, cols_s,
               res_v, sem):
    wid = lax.axis_index("s") * 2 + lax.axis_index("c")

    def popcnt(m):
        return plsc.all_reduce_population_count(m)

    tacc = jnp.zeros((16,), jnp.float32)
    iacc = jnp.zeros((16,), jnp.int32)
    for j in range(_RPS):
        r = wid * _RPS + j
        pltpu.sync_copy(cmf_ref.at[pl.ds(r * _NCH, _NCH)], cm_v)
        pltpu.sync_copy(t32f_ref.at[pl.ds(r * 128, 16)], t32_v)
        t32v = t32_v[...]

        for v in range(_MAXC // 16):
            gidx_v[pl.ds(v * 16, 16)] = jnp.full((16,), r * _NCH, jnp.int32)
            lidx_v[pl.ds(v * 16, 16)] = jnp.zeros((16,), jnp.int32)
        for v in range(_CAND // 16):
            vals_s[pl.ds(v * 16, 16)] = jnp.full((16,), -1.0, jnp.float32)
            cols_s[pl.ds(v * 16, 16)] = jnp.full((16,), 1 << 20, jnp.int32)

        def cbody(v, c):
            m = cm_v[pl.ds(v * 16, 16)] >= t32v
            ids = v * 16 + _iota16()
            cs = plsc.cumsum(jnp.where(m, _splat(1), _splat(0)))
            pos = jnp.minimum(c + cs - 1, _MAXC - 1)
            plsc.store_scatter(lidx_v, [pos], ids, mask=m)
            plsc.store_scatter(gidx_v, [pos], ids + r * _NCH, mask=m)
            return jnp.minimum(c + popcnt(m), _MAXC)

        nc = lax.fori_loop(0, _NCH // 16, cbody, jnp.zeros((16,), jnp.int32))

        pltpu.async_copy(sc2_ref.at[gidx_v], rows_v, sem).wait()
        pltpu.async_copy(ctab_ref.at[lidx_v], cols_v, sem).wait()

        def fbody(s, ns):
            live = jnp.full((16,), s, jnp.int32) < nc
            for j8 in range(_CH // 16):
                v16 = rows_v[s, pl.ds(j8 * 16, 16)]
                c16 = cols_v[s, pl.ds(j8 * 16, 16)]
                m = (v16 >= t32v) & live
                cs = plsc.cumsum(jnp.where(m, _splat(1), _splat(0)))
                pos = jnp.minimum(ns + cs - 1, _CAND - 1)
                plsc.store_scatter(vals_s, [pos], v16, mask=m)
                plsc.store_scatter(cols_s, [pos], c16, mask=m)
                ns = jnp.minimum(ns + popcnt(m), _CAND - 16)
            return ns

        lax.fori_loop(0, _MAXC, fbody, jnp.zeros((16,), jnp.int32))

        def count_ge(cand_splat_i32, strict):
            cv = plsc.bitcast(cand_splat_i32, jnp.float32)
            acc = jnp.zeros((16,), jnp.int32)
            for u in range(_CAND // 16):
                v16 = vals_s[pl.ds(u * 16, 16)]
                m = (v16 > cv) if strict else (v16 >= cv)
                acc = acc + popcnt(m)
            return acc

        def vbody(k, p):
            bit = lax.shift_left(_splat(1), jnp.full((16,), 30 - k, jnp.int32))
            cand = p | bit
            return jnp.where(count_ge(cand, False) >= _K, cand, p)

        t_int = lax.fori_loop(0, 31, vbody, jnp.zeros((16,), jnp.int32))
        t_fv = plsc.bitcast(t_int, jnp.float32)
        quota = _K - count_ge(t_int, True)

        def ibody(k, p):
            bit = lax.shift_left(_splat(1), jnp.full((16,), 14 - k, jnp.int32))
            cand = p | bit
            acc = jnp.zeros((16,), jnp.int32)
            for u in range(_CAND // 16):
                m = (vals_s[pl.ds(u * 16, 16)] == t_fv) & \
                    (cols_s[pl.ds(u * 16, 16)] < cand)
                acc = acc + popcnt(m)
            return jnp.where(acc < quota, cand, p)

        idx_t = lax.fori_loop(0, 15, ibody, jnp.zeros((16,), jnp.int32))

        lane = _iota16() == j
        tacc = jnp.where(lane, t_fv, tacc)
        iacc = jnp.where(lane, idx_t, iacc)

    res_v[...] = tacc
    pltpu.sync_copy(res_v, tout_ref.at[wid])
    res_v[...] = plsc.bitcast(iacc, jnp.float32)
    pltpu.sync_copy(res_v, iout_ref.at[wid])


@functools.partial(
    pl.kernel,
    out_type=[
        jax.ShapeDtypeStruct((_NSC, 16), jnp.float32),
        jax.ShapeDtypeStruct((_NSC, 16), jnp.float32),
    ],
    mesh=plsc.VectorSubcoreMesh(core_axis_name="c", subcore_axis_name="s"),
    compiler_params=pltpu.CompilerParams(needs_layout_passes=False),
    scratch_types=[
        pltpu.VMEM((_NCH,), jnp.float32),
        pltpu.VMEM((16,), jnp.float32),
        pltpu.VMEM((_MAXC,), jnp.int32),
        pltpu.VMEM((_MAXC,), jnp.int32),
        pltpu.VMEM((_MAXC, _CH), jnp.float32),
        pltpu.VMEM((_MAXC, _CH), jnp.int32),
        pltpu.VMEM((_CAND,), jnp.float32),
        pltpu.VMEM((_CAND,), jnp.int32),
        pltpu.VMEM((16,), jnp.float32),
        pltpu.SemaphoreType.DMA,
    ],
)
def _sc_kernel(sc2, cmf, t32f, ctab, tout, iout, *scratch):
    _sc_select(sc2, cmf, t32f, ctab, tout, iout, *scratch)


def kernel(x, W, b, health):
    b2 = b.reshape(1, _N)
    h2 = health.reshape(1, _N)
    scores, t32, cm = pl.pallas_call(
        _score_kernel,
        grid=(_NT,),
        in_specs=[
            pl.BlockSpec((_B, _D), lambda i: (0, 0)),
            pl.BlockSpec((_TN, _D), lambda i: (i, 0)),
            pl.BlockSpec((1, _TN), lambda i: (0, i)),
            pl.BlockSpec((1, _TN), lambda i: (0, i)),
        ],
        out_specs=[
            pl.BlockSpec((_B, _TN), lambda i: (0, i)),
            pl.BlockSpec((_B, 128), lambda i: (0, 0)),
            pl.BlockSpec((1, _B, _CPT), lambda i: (i, 0, 0)),
        ],
        out_shape=[
            jax.ShapeDtypeStruct((_B, _N), jnp.float32),
            jax.ShapeDtypeStruct((_B, 128), jnp.float32),
            jax.ShapeDtypeStruct((_NT, _B, _CPT), jnp.float32),
        ],
        scratch_shapes=[pltpu.VMEM((_NT, _B, _CPT), jnp.float32)],
    )(x, W, b2, h2)

    ctab = (jnp.arange(_NCH, dtype=jnp.int32)[:, None] * _CH
            + jnp.arange(_CH, dtype=jnp.int32)[None, :])
    cmf = jnp.transpose(cm, (1, 0, 2)).reshape(_B * _NCH)
    tsc, isc = _sc_kernel(
        scores.reshape(_B * _NCH, _CH),
        cmf,
        t32.reshape(_B * 128),
        ctab,
    )
    tval = jnp.broadcast_to(tsc[:, :_RPS].reshape(_B, 1), (_B, 128))
    tidx = jnp.broadcast_to(
        lax.bitcast_convert_type(isc[:, :_RPS].reshape(_B, 1), jnp.int32),
        (_B, 128))

    return pl.pallas_call(
        _mask_kernel,
        grid=(_NT,),
        in_specs=[
            pl.BlockSpec((_B, _D), lambda i: (0, 0)),
            pl.BlockSpec((_TN, _D), lambda i: (i, 0)),
            pl.BlockSpec((1, _TN), lambda i: (0, i)),
            pl.BlockSpec((1, _TN), lambda i: (0, i)),
            pl.BlockSpec((_B, 128), lambda i: (0, 0)),
            pl.BlockSpec((_B, 128), lambda i: (0, 0)),
        ],
        out_specs=pl.BlockSpec((_B, _TN), lambda i: (0, i)),
        out_shape=jax.ShapeDtypeStruct((_B, _N), jnp.float32),
    )(x, W, b2, h2, tval, tidx)


# speedup vs baseline: 1.0007x; 1.0007x over previous
import functools

import jax
import jax.numpy as jnp
from jax import lax
from jax.experimental import pallas as pl
from jax.experimental.pallas import tpu as pltpu
from jax.experimental.pallas import tpu_sc as plsc

_B = 128
_D = 64
_N = 32768
_K = 32
_NT = 16
_TN = _N // _NT
_CH = 128
_NCH = _N // _CH
_CPT = _TN // _CH
_MAXC = 48
_CAND = 128
_NSC = 32
_RPS = _B // _NSC
_SENT = 1 << 20


def _scores_tile(x_ref, w_ref, b_ref, h_ref):
    xw = lax.dot_general(x_ref[:], w_ref[:], (((1,), (1,)), ((), ())),
                         preferred_element_type=jnp.float32)
    sig = 1.0 / (1.0 + jnp.exp(-h_ref[:]))
    return jnp.maximum(xw + b_ref[:], 0.0) * sig


def _score_kernel(x_ref, w_ref, b_ref, h_ref, s_out, t32_ref, cm_out, cm_ref):
    i = pl.program_id(0)
    s = _scores_tile(x_ref, w_ref, b_ref, h_ref)
    s_out[:] = s
    mx = jnp.max(s.reshape(_B, _CPT, _CH), axis=2)
    cm_ref[i] = mx
    cm_out[0] = mx

    @pl.when(i == _NT - 1)
    def _():
        def body(k, p):
            cand = p | (1 << (30 - k))
            cf = lax.bitcast_convert_type(cand, jnp.float32)
            cnt = jnp.sum((cm_ref[:] >= cf[None]).astype(jnp.int32),
                          axis=(0, 2))[:, None]
            return jnp.where(cnt >= _K, cand, p)

        t_int = lax.fori_loop(0, 31, body, jnp.zeros((_B, 1), jnp.int32))
        t32_ref[:] = jnp.broadcast_to(
            lax.bitcast_convert_type(t_int, jnp.float32), (_B, 128))


def _iota16():
    return lax.broadcasted_iota(jnp.int32, (16,), 0)


def _splat(v, dt=jnp.int32):
    return jnp.full((16,), v, dt)


def _sc_filter(sc2_ref, cmf_ref, t32f_ref, ctab_ref, vout_ref, cout_ref,
               cm_v, t32_v, gidx_v, lidx_v, rows_v, cols_v, vals_s, cols_s,
               sem):
    wid = lax.axis_index("s") * 2 + lax.axis_index("c")

    for j in range(_RPS):
        r = wid * _RPS + j
        pltpu.sync_copy(cmf_ref.at[pl.ds(r * _NCH, _NCH)], cm_v)
        pltpu.sync_copy(t32f_ref.at[pl.ds(r * 128, 16)], t32_v)
        t32v = t32_v[...]

        for v in range(_MAXC // 16):
            gidx_v[pl.ds(v * 16, 16)] = jnp.full((16,), r * _NCH, jnp.int32)
            lidx_v[pl.ds(v * 16, 16)] = jnp.zeros((16,), jnp.int32)
        for v in range(_CAND // 16):
            vals_s[pl.ds(v * 16, 16)] = jnp.full((16,), -1.0, jnp.float32)
            cols_s[pl.ds(v * 16, 16)] = jnp.full((16,), _SENT, jnp.int32)

        def cbody(v, c):
            m = cm_v[pl.ds(v * 16, 16)] >= t32v
            ids = v * 16 + _iota16()
            cs = plsc.cumsum(jnp.where(m, _splat(1), _splat(0)))
            pos = jnp.minimum(c + cs - 1, _MAXC - 1)
            plsc.store_scatter(lidx_v, [pos], ids, mask=m)
            plsc.store_scatter(gidx_v, [pos], ids + r * _NCH, mask=m)
            return jnp.minimum(c + plsc.all_reduce_population_count(m), _MAXC)

        nc = lax.fori_loop(0, _NCH // 16, cbody, jnp.zeros((16,), jnp.int32))

        pltpu.async_copy(sc2_ref.at[gidx_v], rows_v, sem).wait()
        pltpu.async_copy(ctab_ref.at[lidx_v], cols_v, sem).wait()

        def fbody(s, ns):
            live = jnp.full((16,), s, jnp.int32) < nc
            for j8 in range(_CH // 16):
                v16 = rows_v[s, pl.ds(j8 * 16, 16)]
                c16 = cols_v[s, pl.ds(j8 * 16, 16)]
                m = (v16 >= t32v) & live
                cs = plsc.cumsum(jnp.where(m, _splat(1), _splat(0)))
                pos = jnp.minimum(ns + cs - 1, _CAND - 1)
                plsc.store_scatter(vals_s, [pos], v16, mask=m)
                plsc.store_scatter(cols_s, [pos], c16, mask=m)
                ns = jnp.minimum(ns + plsc.all_reduce_population_count(m),
                                 _CAND - 16)
            return ns

        lax.fori_loop(0, _MAXC, fbody, jnp.zeros((16,), jnp.int32))

        pltpu.sync_copy(vals_s, vout_ref.at[pl.ds(r * _CAND, _CAND)])
        pltpu.sync_copy(cols_s, cout_ref.at[pl.ds(r * _CAND, _CAND)])


@functools.partial(
    pl.kernel,
    out_type=[
        jax.ShapeDtypeStruct((_B * _CAND,), jnp.float32),
        jax.ShapeDtypeStruct((_B * _CAND,), jnp.int32),
    ],
    mesh=plsc.VectorSubcoreMesh(core_axis_name="c", subcore_axis_name="s"),
    compiler_params=pltpu.CompilerParams(needs_layout_passes=False),
    scratch_types=[
        pltpu.VMEM((_NCH,), jnp.float32),
        pltpu.VMEM((16,), jnp.float32),
        pltpu.VMEM((_MAXC,), jnp.int32),
        pltpu.VMEM((_MAXC,), jnp.int32),
        pltpu.VMEM((_MAXC, _CH), jnp.float32),
        pltpu.VMEM((_MAXC, _CH), jnp.int32),
        pltpu.VMEM((_CAND,), jnp.float32),
        pltpu.VMEM((_CAND,), jnp.int32),
        pltpu.SemaphoreType.DMA,
    ],
)
def _sc_kernel(sc2, cmf, t32f, ctab, vout, cout, *scratch):
    _sc_filter(sc2, cmf, t32f, ctab, vout, cout, *scratch)


def _mask_kernel(x_ref, w_ref, b_ref, h_ref, vals_ref, cols_ref, o_ref,
                 tv_scr, ti_scr):
    i = pl.program_id(0)

    @pl.when(i == 0)
    def _():
        def body(k, p):
            cand = p | (1 << (30 - k))
            cf = lax.bitcast_convert_type(cand, jnp.float32)
            cnt = jnp.sum((vals_ref[:] >= cf).astype(jnp.int32),
                          axis=1, keepdims=True)
            return jnp.where(cnt >= _K, cand, p)

        t_int = lax.fori_loop(0, 31, body, jnp.zeros((_B, 1), jnp.int32))
        t_f = lax.bitcast_convert_type(t_int, jnp.float32)
        n_gt = jnp.sum((vals_ref[:] > t_f).astype(jnp.int32),
                       axis=1, keepdims=True)
        quota = _K - n_gt

        def ibody(k, p):
            cand = p | (1 << (14 - k))
            g = jnp.sum(((vals_ref[:] == t_f) &
                         (cols_ref[:] < cand)).astype(jnp.int32),
                        axis=1, keepdims=True)
            return jnp.where(g < quota, cand, p)

        idx_t = lax.fori_loop(0, 15, ibody, jnp.zeros((_B, 1), jnp.int32))
        tv_scr[:] = t_f
        ti_scr[:] = idx_t

    s = _scores_tile(x_ref, w_ref, b_ref, h_ref)
    t = tv_scr[:]
    it = ti_scr[:]
    col = i * _TN + lax.broadcasted_iota(jnp.int32, (_B, _TN), 1)
    keep = (s > t) | ((s == t) & (col <= it))
    o_ref[:] = jnp.where(keep, s, 0.0)


def kernel(x, W, b, health):
    b2 = b.reshape(1, _N)
    h2 = health.reshape(1, _N)
    scores, t32, cm = pl.pallas_call(
        _score_kernel,
        grid=(_NT,),
        in_specs=[
            pl.BlockSpec((_B, _D), lambda i: (0, 0)),
            pl.BlockSpec((_TN, _D), lambda i: (i, 0)),
            pl.BlockSpec((1, _TN), lambda i: (0, i)),
            pl.BlockSpec((1, _TN), lambda i: (0, i)),
        ],
        out_specs=[
            pl.BlockSpec((_B, _TN), lambda i: (0, i)),
            pl.BlockSpec((_B, 128), lambda i: (0, 0)),
            pl.BlockSpec((1, _B, _CPT), lambda i: (i, 0, 0)),
        ],
        out_shape=[
            jax.ShapeDtypeStruct((_B, _N), jnp.float32),
            jax.ShapeDtypeStruct((_B, 128), jnp.float32),
            jax.ShapeDtypeStruct((_NT, _B, _CPT), jnp.float32),
        ],
        scratch_shapes=[pltpu.VMEM((_NT, _B, _CPT), jnp.float32)],
    )(x, W, b2, h2)

    ctab = jnp.arange(_NCH, dtype=jnp.int32)[:, None] * _CH \
        + jnp.arange(_CH, dtype=jnp.int32)[None, :]
    cmf = jnp.transpose(cm, (1, 0, 2)).reshape(_B * _NCH)
    vout, cout = _sc_kernel(
        scores.reshape(_B * _NCH, _CH),
        cmf,
        t32.reshape(_B * 128),
        ctab,
    )

    return pl.pallas_call(
        _mask_kernel,
        grid=(_NT,),
        in_specs=[
            pl.BlockSpec((_B, _D), lambda i: (0, 0)),
            pl.BlockSpec((_TN, _D), lambda i: (i, 0)),
            pl.BlockSpec((1, _TN), lambda i: (0, i)),
            pl.BlockSpec((1, _TN), lambda i: (0, i)),
            pl.BlockSpec((_B, _CAND), lambda i: (0, 0)),
            pl.BlockSpec((_B, _CAND), lambda i: (0, 0)),
        ],
        out_specs=pl.BlockSpec((_B, _TN), lambda i: (0, i)),
        out_shape=jax.ShapeDtypeStruct((_B, _N), jnp.float32),
        scratch_shapes=[pltpu.VMEM((_B, 1), jnp.float32),
                        pltpu.VMEM((_B, 1), jnp.int32)],
    )(x, W, b2, h2, vout.reshape(_B, _CAND), cout.reshape(_B, _CAND))


# speedup vs baseline: 1.0799x; 1.0792x over previous
import functools

import jax
import jax.numpy as jnp
from jax import lax
from jax.experimental import pallas as pl
from jax.experimental.pallas import tpu as pltpu
from jax.experimental.pallas import tpu_sc as plsc

_B = 128
_D = 64
_N = 32768
_K = 32
_NT = 16
_TN = _N // _NT
_CH = 128
_NCH = _N // _CH
_CPT = _TN // _CH
_MAXC = 48
_NSC = 32
_RPS = _B // _NSC


def _scores_tile(x_ref, w_ref, b_ref, h_ref):
    xw = lax.dot_general(x_ref[:], w_ref[:], (((1,), (1,)), ((), ())),
                         preferred_element_type=jnp.float32)
    sig = 1.0 / (1.0 + jnp.exp(-h_ref[:]))
    return jnp.maximum(xw + b_ref[:], 0.0) * sig


def _score_kernel(x_ref, w_ref, b_ref, h_ref, s_out, t32_ref, cm_out, cm_ref):
    i = pl.program_id(0)
    s = _scores_tile(x_ref, w_ref, b_ref, h_ref)
    s_out[:] = s
    mx = jnp.max(s.reshape(_B, _CPT, _CH), axis=2)
    cm_ref[i] = mx
    cm_out[0] = mx

    @pl.when(i == _NT - 1)
    def _():
        def body(k, p):
            cand = p | (1 << (30 - k))
            cf = lax.bitcast_convert_type(cand, jnp.float32)
            cnt = jnp.sum((cm_ref[:] >= cf[None]).astype(jnp.int32),
                          axis=(0, 2))[:, None]
            return jnp.where(cnt >= _K, cand, p)

        t_int = lax.fori_loop(0, 31, body, jnp.zeros((_B, 1), jnp.int32))
        t32_ref[:] = jnp.broadcast_to(
            lax.bitcast_convert_type(t_int, jnp.float32), (_B, 128))


def _iota16():
    return lax.broadcasted_iota(jnp.int32, (16,), 0)


def _splat(v, dt=jnp.int32):
    return jnp.full((16,), v, dt)


def _sc_gather(sc2_ref, cmf_ref, t32f_ref, vout_ref, lout_ref,
               cm_v, t32_v, gidx_v, lidx_v, rows_v, sem):
    wid = lax.axis_index("s") * 2 + lax.axis_index("c")

    for j in range(_RPS):
        r = wid * _RPS + j
        pltpu.sync_copy(cmf_ref.at[pl.ds(r * _NCH, _NCH)], cm_v)
        pltpu.sync_copy(t32f_ref.at[pl.ds(r * 128, 16)], t32_v)
        t32v = t32_v[...]

        for v in range(_MAXC // 16):
            gidx_v[pl.ds(v * 16, 16)] = jnp.full((16,), r * _NCH, jnp.int32)
            lidx_v[pl.ds(v * 16, 16)] = jnp.full((16,), _NCH, jnp.int32)

        def cbody(v, c):
            m = cm_v[pl.ds(v * 16, 16)] >= t32v
            ids = v * 16 + _iota16()
            cs = plsc.cumsum(jnp.where(m, _splat(1), _splat(0)))
            pos = jnp.minimum(c + cs - 1, _MAXC - 1)
            plsc.store_scatter(lidx_v, [pos], ids, mask=m)
            plsc.store_scatter(gidx_v, [pos], ids + r * _NCH, mask=m)
            return jnp.minimum(c + plsc.all_reduce_population_count(m), _MAXC)

        lax.fori_loop(0, _NCH // 16, cbody, jnp.zeros((16,), jnp.int32))

        pltpu.async_copy(sc2_ref.at[gidx_v], rows_v, sem).wait()
        pltpu.sync_copy(rows_v, vout_ref.at[pl.ds(r * _MAXC, _MAXC)])
        pltpu.sync_copy(lidx_v, lout_ref.at[pl.ds(r * _MAXC, _MAXC)])


@functools.partial(
    pl.kernel,
    out_type=[
        jax.ShapeDtypeStruct((_B * _MAXC, _CH), jnp.float32),
        jax.ShapeDtypeStruct((_B * _MAXC,), jnp.int32),
    ],
    mesh=plsc.VectorSubcoreMesh(core_axis_name="c", subcore_axis_name="s"),
    compiler_params=pltpu.CompilerParams(needs_layout_passes=False),
    scratch_types=[
        pltpu.VMEM((_NCH,), jnp.float32),
        pltpu.VMEM((16,), jnp.float32),
        pltpu.VMEM((_MAXC,), jnp.int32),
        pltpu.VMEM((_MAXC,), jnp.int32),
        pltpu.VMEM((_MAXC, _CH), jnp.float32),
        pltpu.SemaphoreType.DMA,
    ],
)
def _sc_kernel(sc2, cmf, t32f, vout, lout, *scratch):
    _sc_gather(sc2, cmf, t32f, vout, lout, *scratch)


def _mask_kernel(x_ref, w_ref, b_ref, h_ref, vals_ref, cid_ref, o_ref,
                 tv_scr, ti_scr):
    i = pl.program_id(0)

    @pl.when(i == 0)
    def _():
        cid3 = cid_ref[:]
        valid = cid3 < _NCH
        cols3 = cid3 * _CH + lax.broadcasted_iota(
            jnp.int32, (_B, _MAXC, _CH), 2)

        def body(k, p):
            cand = p | (1 << (30 - k))
            cf = lax.bitcast_convert_type(cand, jnp.float32)
            cnt = jnp.sum(((vals_ref[:] >= cf[:, :, None]) & valid)
                          .astype(jnp.int32), axis=(1, 2))[:, None]
            return jnp.where(cnt >= _K, cand, p)

        t_int = lax.fori_loop(0, 31, body, jnp.zeros((_B, 1), jnp.int32))
        t_f = lax.bitcast_convert_type(t_int, jnp.float32)
        t_f3 = t_f[:, :, None]
        n_gt = jnp.sum(((vals_ref[:] > t_f3) & valid).astype(jnp.int32),
                       axis=(1, 2))[:, None]
        quota = _K - n_gt

        def ibody(k, p):
            cand = p | (1 << (14 - k))
            g = jnp.sum(((vals_ref[:] == t_f3) & valid &
                         (cols3 < cand[:, :, None])).astype(jnp.int32),
                        axis=(1, 2))[:, None]
            return jnp.where(g < quota, cand, p)

        idx_t = lax.fori_loop(0, 15, ibody, jnp.zeros((_B, 1), jnp.int32))
        tv_scr[:] = t_f
        ti_scr[:] = idx_t

    s = _scores_tile(x_ref, w_ref, b_ref, h_ref)
    t = tv_scr[:]
    it = ti_scr[:]
    col = i * _TN + lax.broadcasted_iota(jnp.int32, (_B, _TN), 1)
    keep = (s > t) | ((s == t) & (col <= it))
    o_ref[:] = jnp.where(keep, s, 0.0)


def kernel(x, W, b, health):
    b2 = b.reshape(1, _N)
    h2 = health.reshape(1, _N)
    scores, t32, cm = pl.pallas_call(
        _score_kernel,
        grid=(_NT,),
        in_specs=[
            pl.BlockSpec((_B, _D), lambda i: (0, 0)),
            pl.BlockSpec((_TN, _D), lambda i: (i, 0)),
            pl.BlockSpec((1, _TN), lambda i: (0, i)),
            pl.BlockSpec((1, _TN), lambda i: (0, i)),
        ],
        out_specs=[
            pl.BlockSpec((_B, _TN), lambda i: (0, i)),
            pl.BlockSpec((_B, 128), lambda i: (0, 0)),
            pl.BlockSpec((1, _B, _CPT), lambda i: (i, 0, 0)),
        ],
        out_shape=[
            jax.ShapeDtypeStruct((_B, _N), jnp.float32),
            jax.ShapeDtypeStruct((_B, 128), jnp.float32),
            jax.ShapeDtypeStruct((_NT, _B, _CPT), jnp.float32),
        ],
        scratch_shapes=[pltpu.VMEM((_NT, _B, _CPT), jnp.float32)],
    )(x, W, b2, h2)

    cmf = jnp.transpose(cm, (1, 0, 2)).reshape(_B * _NCH)
    vout, lout = _sc_kernel(
        scores.reshape(_B * _NCH, _CH),
        cmf,
        t32.reshape(_B * 128),
    )

    return pl.pallas_call(
        _mask_kernel,
        grid=(_NT,),
        in_specs=[
            pl.BlockSpec((_B, _D), lambda i: (0, 0)),
            pl.BlockSpec((_TN, _D), lambda i: (i, 0)),
            pl.BlockSpec((1, _TN), lambda i: (0, i)),
            pl.BlockSpec((1, _TN), lambda i: (0, i)),
            pl.BlockSpec((_B, _MAXC, _CH), lambda i: (0, 0, 0)),
            pl.BlockSpec((_B, _MAXC, 1), lambda i: (0, 0, 0)),
        ],
        out_specs=pl.BlockSpec((_B, _TN), lambda i: (0, i)),
        out_shape=jax.ShapeDtypeStruct((_B, _N), jnp.float32),
        scratch_shapes=[pltpu.VMEM((_B, 1), jnp.float32),
                        pltpu.VMEM((_B, 1), jnp.int32)],
    )(x, W, b2, h2, vout.reshape(_B, _MAXC, _CH), lout.reshape(_B, _MAXC, 1))


# speedup vs baseline: 1.2323x; 1.1411x over previous
import functools

import jax
import jax.numpy as jnp
from jax import lax
from jax.experimental import pallas as pl
from jax.experimental.pallas import tpu as pltpu
from jax.experimental.pallas import tpu_sc as plsc

_B = 128
_D = 64
_N = 32768
_K = 32
_NT = 16
_TN = _N // _NT
_CH = 128
_NCH = _N // _CH
_CPT = _TN // _CH
_MAXC = 48
_NSC = 32
_RPS = _B // _NSC


def _scores_tile(x_ref, w_ref, b_ref, h_ref):
    xw = lax.dot_general(x_ref[:], w_ref[:], (((1,), (1,)), ((), ())),
                         preferred_element_type=jnp.float32)
    sig = 1.0 / (1.0 + jnp.exp(-h_ref[:]))
    return jnp.maximum(xw + b_ref[:], 0.0) * sig


def _score_kernel(x_ref, w_ref, b_ref, h_ref, s_out, t32_ref, cm_out, cm_ref):
    i = pl.program_id(0)
    s = _scores_tile(x_ref, w_ref, b_ref, h_ref)
    s_out[:] = s
    mxt = jnp.transpose(jnp.max(s.reshape(_B, _CPT, _CH), axis=2))
    cm_ref[i] = mxt
    cm_out[0] = mxt

    @pl.when(i == _NT - 1)
    def _():
        def body(k, p):
            cand = p | (1 << (30 - k))
            cf = lax.bitcast_convert_type(cand, jnp.float32)
            cnt = jnp.sum((cm_ref[:] >= cf[None]).astype(jnp.int32),
                          axis=(0, 1))[None, :]
            return jnp.where(cnt >= _K, cand, p)

        t_int = lax.fori_loop(0, 31, body, jnp.zeros((1, _B), jnp.int32))
        t_f = lax.bitcast_convert_type(t_int, jnp.float32)
        t32_ref[:] = jnp.broadcast_to(jnp.transpose(t_f), (_B, 128))


def _iota16():
    return lax.broadcasted_iota(jnp.int32, (16,), 0)


def _splat(v, dt=jnp.int32):
    return jnp.full((16,), v, dt)


def _sc_gather(sc2_ref, cmf_ref, t32f_ref, vout_ref, lout_ref,
               cm_v, t32_v, gidx_v, lidx_v, rows_v, sem):
    wid = lax.axis_index("s") * 2 + lax.axis_index("c")

    for j in range(_RPS):
        r = wid * _RPS + j
        pltpu.sync_copy(cmf_ref.at[pl.ds(r * _NCH, _NCH)], cm_v)
        pltpu.sync_copy(t32f_ref.at[pl.ds(r * 128, 16)], t32_v)
        t32v = t32_v[...]

        for v in range(_MAXC // 16):
            gidx_v[pl.ds(v * 16, 16)] = jnp.full((16,), r * _NCH, jnp.int32)
            lidx_v[pl.ds(v * 16, 16)] = jnp.full((16,), _NCH, jnp.int32)

        def cbody(v, c):
            m = cm_v[pl.ds(v * 16, 16)] >= t32v
            ids = v * 16 + _iota16()
            cs = plsc.cumsum(jnp.where(m, _splat(1), _splat(0)))
            pos = jnp.minimum(c + cs - 1, _MAXC - 1)
            plsc.store_scatter(lidx_v, [pos], ids, mask=m)
            plsc.store_scatter(gidx_v, [pos], ids + r * _NCH, mask=m)
            return jnp.minimum(c + plsc.all_reduce_population_count(m), _MAXC)

        lax.fori_loop(0, _NCH // 16, cbody, jnp.zeros((16,), jnp.int32))

        pltpu.async_copy(sc2_ref.at[gidx_v], rows_v, sem).wait()
        pltpu.sync_copy(rows_v, vout_ref.at[pl.ds(r * _MAXC, _MAXC)])
        pltpu.sync_copy(lidx_v, lout_ref.at[pl.ds(r * _MAXC, _MAXC)])


@functools.partial(
    pl.kernel,
    out_type=[
        jax.ShapeDtypeStruct((_B * _MAXC, _CH), jnp.float32),
        jax.ShapeDtypeStruct((_B * _MAXC,), jnp.int32),
    ],
    mesh=plsc.VectorSubcoreMesh(core_axis_name="c", subcore_axis_name="s"),
    compiler_params=pltpu.CompilerParams(needs_layout_passes=False),
    scratch_types=[
        pltpu.VMEM((_NCH,), jnp.float32),
        pltpu.VMEM((16,), jnp.float32),
        pltpu.VMEM((_MAXC,), jnp.int32),
        pltpu.VMEM((_MAXC,), jnp.int32),
        pltpu.VMEM((_MAXC, _CH), jnp.float32),
        pltpu.SemaphoreType.DMA,
    ],
)
def _sc_kernel(sc2, cmf, t32f, vout, lout, *scratch):
    _sc_gather(sc2, cmf, t32f, vout, lout, *scratch)


def _mask_kernel(x_ref, w_ref, b_ref, h_ref, vals_ref, cid_ref, t32_ref,
                 o_ref, tv_scr, ti_scr, wk_scr, tc_scr):
    i = pl.program_id(0)

    @pl.when(i == 0)
    def _():
        cid3 = cid_ref[:]
        t32r = t32_ref[:, 0:1][:, :, None]
        v = vals_ref[:]
        wk_scr[:] = jnp.where((cid3 < _NCH) & (v >= t32r), v, -1.0)

        def count_ge(cf):
            return jnp.sum((wk_scr[:] >= cf[:, :, None]).astype(jnp.int32),
                           axis=(1, 2))[:, None]

        def body(k, p):
            b1 = 1 << (30 - 2 * k)
            b0 = b1 >> 1
            c1, c2, c3 = p | b1, p | b0, p | b1 | b0
            n1 = count_ge(lax.bitcast_convert_type(c1, jnp.float32))
            n2 = count_ge(lax.bitcast_convert_type(c2, jnp.float32))
            n3 = count_ge(lax.bitcast_convert_type(c3, jnp.float32))
            return jnp.where(n1 >= _K, jnp.where(n3 >= _K, c3, c1),
                             jnp.where(n2 >= _K, c2, p))

        p = lax.fori_loop(0, 15, body, jnp.zeros((_B, 1), jnp.int32))
        c1 = p | 1
        n1 = count_ge(lax.bitcast_convert_type(c1, jnp.float32))
        t_int = jnp.where(n1 >= _K, c1, p)
        t_f = lax.bitcast_convert_type(t_int, jnp.float32)
        t_f3 = t_f[:, :, None]
        n_gt = jnp.sum((wk_scr[:] > t_f3).astype(jnp.int32),
                       axis=(1, 2))[:, None]
        quota = _K - n_gt

        cols3 = cid3 * _CH + lax.broadcasted_iota(
            jnp.int32, (_B, _MAXC, _CH), 2)
        tc_scr[:] = jnp.where(wk_scr[:] == t_f3, cols3, jnp.int32(1 << 20))

        def count_lt(c):
            return jnp.sum((tc_scr[:] < c[:, :, None]).astype(jnp.int32),
                           axis=(1, 2))[:, None]

        def ibody(k, p):
            b1 = 1 << (14 - 2 * k)
            b0 = b1 >> 1
            c1, c2, c3 = p | b1, p | b0, p | b1 | b0
            g1, g2, g3 = count_lt(c1), count_lt(c2), count_lt(c3)
            return jnp.where(g1 < quota, jnp.where(g3 < quota, c3, c1),
                             jnp.where(g2 < quota, c2, p))

        p2 = lax.fori_loop(0, 7, ibody, jnp.zeros((_B, 1), jnp.int32))
        c1b = p2 | 1
        idx_t = jnp.where(count_lt(c1b) < quota, c1b, p2)
        tv_scr[:] = t_f
        ti_scr[:] = idx_t

    s = _scores_tile(x_ref, w_ref, b_ref, h_ref)
    t = tv_scr[:]
    it = ti_scr[:]
    col = i * _TN + lax.broadcasted_iota(jnp.int32, (_B, _TN), 1)
    keep = (s > t) | ((s == t) & (col <= it))
    o_ref[:] = jnp.where(keep, s, 0.0)


def kernel(x, W, b, health):
    b2 = b.reshape(1, _N)
    h2 = health.reshape(1, _N)
    scores, t32, cm = pl.pallas_call(
        _score_kernel,
        grid=(_NT,),
        in_specs=[
            pl.BlockSpec((_B, _D), lambda i: (0, 0)),
            pl.BlockSpec((_TN, _D), lambda i: (i, 0)),
            pl.BlockSpec((1, _TN), lambda i: (0, i)),
            pl.BlockSpec((1, _TN), lambda i: (0, i)),
        ],
        out_specs=[
            pl.BlockSpec((_B, _TN), lambda i: (0, i)),
            pl.BlockSpec((_B, 128), lambda i: (0, 0)),
            pl.BlockSpec((1, _CPT, _B), lambda i: (i, 0, 0)),
        ],
        out_shape=[
            jax.ShapeDtypeStruct((_B, _N), jnp.float32),
            jax.ShapeDtypeStruct((_B, 128), jnp.float32),
            jax.ShapeDtypeStruct((_NT, _CPT, _B), jnp.float32),
        ],
        scratch_shapes=[pltpu.VMEM((_NT, _CPT, _B), jnp.float32)],
    )(x, W, b2, h2)

    cmf = jnp.transpose(cm, (2, 0, 1)).reshape(_B * _NCH)
    vout, lout = _sc_kernel(
        scores.reshape(_B * _NCH, _CH),
        cmf,
        t32.reshape(_B * 128),
    )

    return pl.pallas_call(
        _mask_kernel,
        grid=(_NT,),
        in_specs=[
            pl.BlockSpec((_B, _D), lambda i: (0, 0)),
            pl.BlockSpec((_TN, _D), lambda i: (i, 0)),
            pl.BlockSpec((1, _TN), lambda i: (0, i)),
            pl.BlockSpec((1, _TN), lambda i: (0, i)),
            pl.BlockSpec((_B, _MAXC, _CH), lambda i: (0, 0, 0)),
            pl.BlockSpec((_B, _MAXC, 1), lambda i: (0, 0, 0)),
            pl.BlockSpec((_B, 128), lambda i: (0, 0)),
        ],
        out_specs=pl.BlockSpec((_B, _TN), lambda i: (0, i)),
        out_shape=jax.ShapeDtypeStruct((_B, _N), jnp.float32),
        scratch_shapes=[pltpu.VMEM((_B, 1), jnp.float32),
                        pltpu.VMEM((_B, 1), jnp.int32),
                        pltpu.VMEM((_B, _MAXC, _CH), jnp.float32),
                        pltpu.VMEM((_B, _MAXC, _CH), jnp.int32)],
    )(x, W, b2, h2, vout.reshape(_B, _MAXC, _CH), lout.reshape(_B, _MAXC, 1),
      t32)


# speedup vs baseline: 1.4065x; 1.1414x over previous
import functools

import jax
import jax.numpy as jnp
from jax import lax
from jax.experimental import pallas as pl
from jax.experimental.pallas import tpu as pltpu
from jax.experimental.pallas import tpu_sc as plsc

_B = 128
_D = 64
_N = 32768
_K = 32
_NT = 16
_TN = _N // _NT
_CH = 128
_NCH = _N // _CH
_CPT = _TN // _CH
_MAXC = 48
_NSC = 32
_RPS = _B // _NSC


def _scores_tile(x_ref, w_ref, b_ref, h_ref):
    xw = lax.dot_general(x_ref[:], w_ref[:], (((1,), (1,)), ((), ())),
                         preferred_element_type=jnp.float32)
    sig = 1.0 / (1.0 + jnp.exp(-h_ref[:]))
    return jnp.maximum(xw + b_ref[:], 0.0) * sig


def _score_kernel(x_ref, w_ref, b_ref, h_ref, s_out, t32_ref, cm_out, cm_ref):
    i = pl.program_id(0)
    s3 = _scores_tile(x_ref, w_ref, b_ref, h_ref).reshape(_B, _CPT, _CH)
    s_out[:] = s3
    mxt = jnp.transpose(jnp.max(s3, axis=2))
    cm_ref[i] = mxt
    cm_out[0] = mxt

    @pl.when(i == _NT - 1)
    def _():
        def body(k, p):
            cand = p | (1 << (30 - k))
            cf = lax.bitcast_convert_type(cand, jnp.float32)
            cnt = jnp.sum((cm_ref[:] >= cf[None]).astype(jnp.int32),
                          axis=(0, 1))[None, :]
            return jnp.where(cnt >= _K, cand, p)

        t_int = lax.fori_loop(0, 31, body, jnp.zeros((1, _B), jnp.int32))
        t_f = lax.bitcast_convert_type(t_int, jnp.float32)
        t32_ref[:] = jnp.broadcast_to(jnp.transpose(t_f), (_B, 128))


def _iota16():
    return lax.broadcasted_iota(jnp.int32, (16,), 0)


def _splat(v, dt=jnp.int32):
    return jnp.full((16,), v, dt)


def _sc_gather(sc2_ref, cmf_ref, t32f_ref, vout_ref, lout_ref,
               cm_v, t32_v, gidx_v, lidx_v, rows_v, sem):
    wid = lax.axis_index("s") * 2 + lax.axis_index("c")

    for j in range(_RPS):
        r = wid * _RPS + j
        pltpu.sync_copy(cmf_ref.at[pl.ds(r * _NCH, _NCH)], cm_v)
        pltpu.sync_copy(t32f_ref.at[pl.ds(r * 128, 16)], t32_v)
        t32v = t32_v[...]

        for v in range(_MAXC // 16):
            gidx_v[pl.ds(v * 16, 16)] = jnp.full((16,), r * _NCH, jnp.int32)
            lidx_v[pl.ds(v * 16, 16)] = jnp.full((16,), _NCH, jnp.int32)

        def cbody(v, c):
            m = cm_v[pl.ds(v * 16, 16)] >= t32v
            ids = v * 16 + _iota16()
            cs = plsc.cumsum(jnp.where(m, _splat(1), _splat(0)))
            pos = jnp.minimum(c + cs - 1, _MAXC - 1)
            plsc.store_scatter(lidx_v, [pos], ids, mask=m)
            plsc.store_scatter(gidx_v, [pos], ids + r * _NCH, mask=m)
            return jnp.minimum(c + plsc.all_reduce_population_count(m), _MAXC)

        lax.fori_loop(0, _NCH // 16, cbody, jnp.zeros((16,), jnp.int32))

        pltpu.async_copy(sc2_ref.at[gidx_v], rows_v, sem).wait()
        pltpu.sync_copy(rows_v, vout_ref.at[pl.ds(r * _MAXC, _MAXC)])
        pltpu.sync_copy(lidx_v, lout_ref.at[pl.ds(r * _MAXC, _MAXC)])


@functools.partial(
    pl.kernel,
    out_type=[
        jax.ShapeDtypeStruct((_B * _MAXC, _CH), jnp.float32),
        jax.ShapeDtypeStruct((_B * _MAXC,), jnp.int32),
    ],
    mesh=plsc.VectorSubcoreMesh(core_axis_name="c", subcore_axis_name="s"),
    compiler_params=pltpu.CompilerParams(needs_layout_passes=False),
    scratch_types=[
        pltpu.VMEM((_NCH,), jnp.float32),
        pltpu.VMEM((16,), jnp.float32),
        pltpu.VMEM((_MAXC,), jnp.int32),
        pltpu.VMEM((_MAXC,), jnp.int32),
        pltpu.VMEM((_MAXC, _CH), jnp.float32),
        pltpu.SemaphoreType.DMA,
    ],
)
def _sc_kernel(sc2, cmf, t32f, vout, lout, *scratch):
    _sc_gather(sc2, cmf, t32f, vout, lout, *scratch)


def _mask_kernel(x_ref, w_ref, b_ref, h_ref, vals_ref, cid_ref, t32_ref,
                 o_ref, tv_scr, ti_scr, wk_scr, tc_scr):
    i = pl.program_id(0)

    @pl.when(i == 0)
    def _():
        cid3 = cid_ref[:]
        t32r = t32_ref[:, 0:1][:, :, None]
        v = vals_ref[:]
        wk_scr[:] = jnp.where((cid3 < _NCH) & (v >= t32r), v, -1.0)

        def count_ge(cf):
            return jnp.sum((wk_scr[:] >= cf[:, :, None]).astype(jnp.int32),
                           axis=(1, 2))[:, None]

        def body(k, p):
            b1 = 1 << (30 - 2 * k)
            b0 = b1 >> 1
            c1, c2, c3 = p | b1, p | b0, p | b1 | b0
            n1 = count_ge(lax.bitcast_convert_type(c1, jnp.float32))
            n2 = count_ge(lax.bitcast_convert_type(c2, jnp.float32))
            n3 = count_ge(lax.bitcast_convert_type(c3, jnp.float32))
            return jnp.where(n1 >= _K, jnp.where(n3 >= _K, c3, c1),
                             jnp.where(n2 >= _K, c2, p))

        p = lax.fori_loop(0, 15, body, jnp.zeros((_B, 1), jnp.int32))
        c1 = p | 1
        n1 = count_ge(lax.bitcast_convert_type(c1, jnp.float32))
        t_int = jnp.where(n1 >= _K, c1, p)
        t_f = lax.bitcast_convert_type(t_int, jnp.float32)
        t_f3 = t_f[:, :, None]
        n_gt = jnp.sum((wk_scr[:] > t_f3).astype(jnp.int32),
                       axis=(1, 2))[:, None]
        quota = _K - n_gt

        cols3 = cid3 * _CH + lax.broadcasted_iota(
            jnp.int32, (_B, _MAXC, _CH), 2)
        tc_scr[:] = jnp.where(wk_scr[:] == t_f3, cols3, jnp.int32(1 << 20))

        def count_lt(c):
            return jnp.sum((tc_scr[:] < c[:, :, None]).astype(jnp.int32),
                           axis=(1, 2))[:, None]

        def ibody(k, p):
            b1 = 1 << (14 - 2 * k)
            b0 = b1 >> 1
            c1, c2, c3 = p | b1, p | b0, p | b1 | b0
            g1, g2, g3 = count_lt(c1), count_lt(c2), count_lt(c3)
            return jnp.where(g1 < quota, jnp.where(g3 < quota, c3, c1),
                             jnp.where(g2 < quota, c2, p))

        p2 = lax.fori_loop(0, 7, ibody, jnp.zeros((_B, 1), jnp.int32))
        c1b = p2 | 1
        idx_t = jnp.where(count_lt(c1b) < quota, c1b, p2)
        tv_scr[:] = t_f
        ti_scr[:] = idx_t

    s = _scores_tile(x_ref, w_ref, b_ref, h_ref)
    t = tv_scr[:]
    it = ti_scr[:]
    col = i * _TN + lax.broadcasted_iota(jnp.int32, (_B, _TN), 1)
    keep = (s > t) | ((s == t) & (col <= it))
    o_ref[:] = jnp.where(keep, s, 0.0)


def kernel(x, W, b, health):
    b2 = b.reshape(1, _N)
    h2 = health.reshape(1, _N)
    scores, t32, cm = pl.pallas_call(
        _score_kernel,
        grid=(_NT,),
        in_specs=[
            pl.BlockSpec((_B, _D), lambda i: (0, 0)),
            pl.BlockSpec((_TN, _D), lambda i: (i, 0)),
            pl.BlockSpec((1, _TN), lambda i: (0, i)),
            pl.BlockSpec((1, _TN), lambda i: (0, i)),
        ],
        out_specs=[
            pl.BlockSpec((_B, _CPT, _CH), lambda i: (0, i, 0)),
            pl.BlockSpec((_B, 128), lambda i: (0, 0)),
            pl.BlockSpec((1, _CPT, _B), lambda i: (i, 0, 0)),
        ],
        out_shape=[
            jax.ShapeDtypeStruct((_B, _NCH, _CH), jnp.float32),
            jax.ShapeDtypeStruct((_B, 128), jnp.float32),
            jax.ShapeDtypeStruct((_NT, _CPT, _B), jnp.float32),
        ],
        scratch_shapes=[pltpu.VMEM((_NT, _CPT, _B), jnp.float32)],
    )(x, W, b2, h2)

    cmf = jnp.transpose(cm, (2, 0, 1)).reshape(_B * _NCH)
    vout, lout = _sc_kernel(
        scores.reshape(_B * _NCH, _CH),
        cmf,
        t32.reshape(_B * 128),
    )

    return pl.pallas_call(
        _mask_kernel,
        grid=(_NT,),
        in_specs=[
            pl.BlockSpec((_B, _D), lambda i: (0, 0)),
            pl.BlockSpec((_TN, _D), lambda i: (i, 0)),
            pl.BlockSpec((1, _TN), lambda i: (0, i)),
            pl.BlockSpec((1, _TN), lambda i: (0, i)),
            pl.BlockSpec((_B, _MAXC, _CH), lambda i: (0, 0, 0)),
            pl.BlockSpec((_B, _MAXC, 1), lambda i: (0, 0, 0)),
            pl.BlockSpec((_B, 128), lambda i: (0, 0)),
        ],
        out_specs=pl.BlockSpec((_B, _TN), lambda i: (0, i)),
        out_shape=jax.ShapeDtypeStruct((_B, _N), jnp.float32),
        scratch_shapes=[pltpu.VMEM((_B, 1), jnp.float32),
                        pltpu.VMEM((_B, 1), jnp.int32),
                        pltpu.VMEM((_B, _MAXC, _CH), jnp.float32),
                        pltpu.VMEM((_B, _MAXC, _CH), jnp.int32)],
    )(x, W, b2, h2, vout.reshape(_B, _MAXC, _CH), lout.reshape(_B, _MAXC, 1),
      t32)


# speedup vs baseline: 1.5186x; 1.0797x over previous
import functools

import jax
import jax.numpy as jnp
from jax import lax
from jax.experimental import pallas as pl
from jax.experimental.pallas import tpu as pltpu
from jax.experimental.pallas import tpu_sc as plsc

_B = 128
_D = 64
_N = 32768
_K = 32
_NT = 16
_TN = _N // _NT
_CH = 128
_NCH = _N // _CH
_CPT = _TN // _CH
_MAXC = 48
_NSC = 32
_RPS = _B // _NSC


def _scores_tile(x_ref, w_ref, b_ref, h_ref):
    xw = lax.dot_general(x_ref[:], w_ref[:], (((1,), (1,)), ((), ())),
                         preferred_element_type=jnp.float32)
    sig = 1.0 / (1.0 + jnp.exp(-h_ref[:]))
    return jnp.maximum(xw + b_ref[:], 0.0) * sig


def _score_kernel(x_ref, w_ref, b_ref, h_ref, s_out, t32_ref, cm_out, cm_ref):
    i = pl.program_id(0)
    s3 = _scores_tile(x_ref, w_ref, b_ref, h_ref).reshape(_B, _CPT, _CH)
    s_out[:] = s3
    mxt = jnp.transpose(jnp.max(s3, axis=2))
    cm_ref[i] = mxt
    cm_out[0] = mxt

    @pl.when(i == _NT - 1)
    def _():
        def body(k, p):
            cand = p | (1 << (30 - k))
            cf = lax.bitcast_convert_type(cand, jnp.float32)
            cnt = jnp.sum((cm_ref[:] >= cf[None]).astype(jnp.int32),
                          axis=(0, 1))[None, :]
            return jnp.where(cnt >= _K, cand, p)

        t_int = lax.fori_loop(0, 31, body, jnp.zeros((1, _B), jnp.int32))
        t_f = lax.bitcast_convert_type(t_int, jnp.float32)
        t32_ref[:] = jnp.broadcast_to(jnp.transpose(t_f), (_B, 128))


def _iota16():
    return lax.broadcasted_iota(jnp.int32, (16,), 0)


def _splat(v, dt=jnp.int32):
    return jnp.full((16,), v, dt)


def _sc_gather(sc2_ref, cmf_ref, t32f_ref, vout_ref, lout_ref,
               cm_v, t32_v, gidx_v, lidx_v, rows_v, sem):
    wid = lax.axis_index("s") * 2 + lax.axis_index("c")

    for j in range(_RPS):
        r = wid * _RPS + j
        pltpu.sync_copy(cmf_ref.at[pl.ds(r * _NCH, _NCH)], cm_v)
        pltpu.sync_copy(t32f_ref.at[pl.ds(r * 128, 16)], t32_v)
        t32v = t32_v[...]

        for v in range(_MAXC // 16):
            gidx_v[pl.ds(v * 16, 16)] = jnp.full((16,), r * _NCH, jnp.int32)
            lidx_v[pl.ds(v * 16, 16)] = jnp.full((16,), _NCH, jnp.int32)

        def cbody(v, c):
            m = cm_v[pl.ds(v * 16, 16)] >= t32v
            ids = v * 16 + _iota16()
            cs = plsc.cumsum(jnp.where(m, _splat(1), _splat(0)))
            pos = jnp.minimum(c + cs - 1, _MAXC - 1)
            plsc.store_scatter(lidx_v, [pos], ids, mask=m)
            plsc.store_scatter(gidx_v, [pos], ids + r * _NCH, mask=m)
            return jnp.minimum(c + plsc.all_reduce_population_count(m), _MAXC)

        lax.fori_loop(0, _NCH // 16, cbody, jnp.zeros((16,), jnp.int32))

        pltpu.async_copy(sc2_ref.at[gidx_v], rows_v, sem).wait()
        pltpu.sync_copy(rows_v, vout_ref.at[pl.ds(r * _MAXC, _MAXC)])
        pltpu.sync_copy(lidx_v, lout_ref.at[pl.ds(r * _MAXC, _MAXC)])


@functools.partial(
    pl.kernel,
    out_type=[
        jax.ShapeDtypeStruct((_B * _MAXC, _CH), jnp.float32),
        jax.ShapeDtypeStruct((_B * _MAXC,), jnp.int32),
    ],
    mesh=plsc.VectorSubcoreMesh(core_axis_name="c", subcore_axis_name="s"),
    compiler_params=pltpu.CompilerParams(needs_layout_passes=False),
    scratch_types=[
        pltpu.VMEM((_NCH,), jnp.float32),
        pltpu.VMEM((16,), jnp.float32),
        pltpu.VMEM((_MAXC,), jnp.int32),
        pltpu.VMEM((_MAXC,), jnp.int32),
        pltpu.VMEM((_MAXC, _CH), jnp.float32),
        pltpu.SemaphoreType.DMA,
    ],
)
def _sc_kernel(sc2, cmf, t32f, vout, lout, *scratch):
    _sc_gather(sc2, cmf, t32f, vout, lout, *scratch)


def _mask_kernel(x_ref, w_ref, b_ref, h_ref, vals_ref, cid_ref, t32_ref,
                 o_ref, tv_scr, ti_scr, wk_scr, tc_scr):
    i = pl.program_id(0)

    @pl.when(i == 0)
    def _():
        cid3 = cid_ref[:]
        t32r = t32_ref[:, 0:1][:, :, None]
        v = vals_ref[:]
        wk_scr[:] = jnp.where((cid3 < _NCH) & (v >= t32r), v,
                              -1.0).reshape(_B, _MAXC * _CH)
        ones = jnp.ones((_MAXC * _CH, 128), jnp.float32)

        def matcnt(ind):
            return lax.dot_general(
                ind.astype(jnp.float32), ones, (((1,), (0,)), ((), ())),
                preferred_element_type=jnp.float32)[:, 0:1]

        def body(k, p):
            cand = p | (1 << (30 - k))
            cf = lax.bitcast_convert_type(cand, jnp.float32)
            cnt = matcnt(wk_scr[:] >= cf)
            return jnp.where(cnt >= _K, cand, p)

        t_int = lax.fori_loop(0, 31, body, jnp.zeros((_B, 1), jnp.int32))
        t_f = lax.bitcast_convert_type(t_int, jnp.float32)
        n_gt = matcnt(wk_scr[:] > t_f)
        quota = _K - n_gt

        cols3 = cid3 * _CH + lax.broadcasted_iota(
            jnp.int32, (_B, _MAXC, _CH), 2)
        tc_scr[:] = jnp.where(wk_scr[:].reshape(_B, _MAXC, _CH) == t_f[:, :, None],
                              cols3, jnp.int32(1 << 20)).reshape(
                                  _B, _MAXC * _CH)

        def ibody(k, p):
            cand = p | (1 << (14 - k))
            g = matcnt(tc_scr[:] < cand)
            return jnp.where(g < quota, cand, p)

        idx_t = lax.fori_loop(0, 15, ibody, jnp.zeros((_B, 1), jnp.int32))
        tv_scr[:] = t_f
        ti_scr[:] = idx_t

    s = _scores_tile(x_ref, w_ref, b_ref, h_ref)
    t = tv_scr[:]
    it = ti_scr[:]
    col = i * _TN + lax.broadcasted_iota(jnp.int32, (_B, _TN), 1)
    keep = (s > t) | ((s == t) & (col <= it))
    o_ref[:] = jnp.where(keep, s, 0.0)


def kernel(x, W, b, health):
    b2 = b.reshape(1, _N)
    h2 = health.reshape(1, _N)
    scores, t32, cm = pl.pallas_call(
        _score_kernel,
        grid=(_NT,),
        in_specs=[
            pl.BlockSpec((_B, _D), lambda i: (0, 0)),
            pl.BlockSpec((_TN, _D), lambda i: (i, 0)),
            pl.BlockSpec((1, _TN), lambda i: (0, i)),
            pl.BlockSpec((1, _TN), lambda i: (0, i)),
        ],
        out_specs=[
            pl.BlockSpec((_B, _CPT, _CH), lambda i: (0, i, 0)),
            pl.BlockSpec((_B, 128), lambda i: (0, 0)),
            pl.BlockSpec((1, _CPT, _B), lambda i: (i, 0, 0)),
        ],
        out_shape=[
            jax.ShapeDtypeStruct((_B, _NCH, _CH), jnp.float32),
            jax.ShapeDtypeStruct((_B, 128), jnp.float32),
            jax.ShapeDtypeStruct((_NT, _CPT, _B), jnp.float32),
        ],
        scratch_shapes=[pltpu.VMEM((_NT, _CPT, _B), jnp.float32)],
    )(x, W, b2, h2)

    cmf = jnp.transpose(cm, (2, 0, 1)).reshape(_B * _NCH)
    vout, lout = _sc_kernel(
        scores.reshape(_B * _NCH, _CH),
        cmf,
        t32.reshape(_B * 128),
    )

    return pl.pallas_call(
        _mask_kernel,
        grid=(_NT,),
        in_specs=[
            pl.BlockSpec((_B, _D), lambda i: (0, 0)),
            pl.BlockSpec((_TN, _D), lambda i: (i, 0)),
            pl.BlockSpec((1, _TN), lambda i: (0, i)),
            pl.BlockSpec((1, _TN), lambda i: (0, i)),
            pl.BlockSpec((_B, _MAXC, _CH), lambda i: (0, 0, 0)),
            pl.BlockSpec((_B, _MAXC, 1), lambda i: (0, 0, 0)),
            pl.BlockSpec((_B, 128), lambda i: (0, 0)),
        ],
        out_specs=pl.BlockSpec((_B, _TN), lambda i: (0, i)),
        out_shape=jax.ShapeDtypeStruct((_B, _N), jnp.float32),
        scratch_shapes=[pltpu.VMEM((_B, 1), jnp.float32),
                        pltpu.VMEM((_B, 1), jnp.int32),
                        pltpu.VMEM((_B, _MAXC * _CH), jnp.float32),
                        pltpu.VMEM((_B, _MAXC * _CH), jnp.int32)],
    )(x, W, b2, h2, vout.reshape(_B, _MAXC, _CH), lout.reshape(_B, _MAXC, 1),
      t32)


# speedup vs baseline: 1.6577x; 1.0916x over previous
import functools

import jax
import jax.numpy as jnp
from jax import lax
from jax.experimental import pallas as pl
from jax.experimental.pallas import tpu as pltpu
from jax.experimental.pallas import tpu_sc as plsc

_B = 128
_D = 64
_N = 32768
_K = 32
_NT = 16
_TN = _N // _NT
_CH = 128
_NCH = _N // _CH
_CPT = _TN // _CH
_MAXC = 40
_NSC = 32
_RPS = _B // _NSC


def _scores_tile(x_ref, w_ref, b_ref, h_ref):
    xw = lax.dot_general(x_ref[:], w_ref[:], (((1,), (1,)), ((), ())),
                         preferred_element_type=jnp.float32)
    sig = 1.0 / (1.0 + jnp.exp(-h_ref[:]))
    return jnp.maximum(xw + b_ref[:], 0.0) * sig


def _score_kernel(x_ref, w_ref, b_ref, h_ref, s_out, t32_ref, cm_out, cm_ref):
    i = pl.program_id(0)
    s3 = _scores_tile(x_ref, w_ref, b_ref, h_ref).reshape(_B, _CPT, _CH)
    s_out[:] = s3
    mxt = jnp.transpose(jnp.max(s3, axis=2))
    cm_ref[i] = mxt
    cm_out[0] = mxt

    @pl.when(i == _NT - 1)
    def _():
        def body(k, p):
            cand = p | (1 << (30 - k))
            cf = lax.bitcast_convert_type(cand, jnp.float32)
            cnt = jnp.sum((cm_ref[:] >= cf[None]).astype(jnp.int32),
                          axis=(0, 1))[None, :]
            return jnp.where(cnt >= _K, cand, p)

        t_int = lax.fori_loop(0, 31, body, jnp.zeros((1, _B), jnp.int32))
        t_f = lax.bitcast_convert_type(t_int, jnp.float32)
        t32_ref[:] = jnp.broadcast_to(jnp.transpose(t_f), (_B, 128))


def _iota16():
    return lax.broadcasted_iota(jnp.int32, (16,), 0)


def _splat(v, dt=jnp.int32):
    return jnp.full((16,), v, dt)


def _sc_gather(sc2_ref, cmf_ref, t32f_ref, vout_ref, lout_ref,
               cm_v, t32_v, gidx_v, lidx_v, rows_v, sem):
    wid = lax.axis_index("s") * 2 + lax.axis_index("c")

    for j in range(_RPS):
        r = wid * _RPS + j
        pltpu.sync_copy(cmf_ref.at[pl.ds(r * _NCH, _NCH)], cm_v)
        pltpu.sync_copy(t32f_ref.at[pl.ds(r * 128, 16)], t32_v)
        t32v = t32_v[...]

        for off in (0, 16, _MAXC - 16):
            gidx_v[pl.ds(off, 16)] = jnp.full((16,), r * _NCH, jnp.int32)
            lidx_v[pl.ds(off, 16)] = jnp.full((16,), _NCH, jnp.int32)

        def cbody(v, c):
            m = cm_v[pl.ds(v * 16, 16)] >= t32v
            ids = v * 16 + _iota16()
            cs = plsc.cumsum(jnp.where(m, _splat(1), _splat(0)))
            pos = jnp.minimum(c + cs - 1, _MAXC - 1)
            plsc.store_scatter(lidx_v, [pos], ids, mask=m)
            plsc.store_scatter(gidx_v, [pos], ids + r * _NCH, mask=m)
            return jnp.minimum(c + plsc.all_reduce_population_count(m), _MAXC)

        lax.fori_loop(0, _NCH // 16, cbody, jnp.zeros((16,), jnp.int32))

        pltpu.async_copy(sc2_ref.at[gidx_v], rows_v, sem).wait()
        pltpu.sync_copy(rows_v, vout_ref.at[pl.ds(r * _MAXC, _MAXC)])
        pltpu.sync_copy(lidx_v, lout_ref.at[pl.ds(r * _MAXC, _MAXC)])


@functools.partial(
    pl.kernel,
    out_type=[
        jax.ShapeDtypeStruct((_B * _MAXC, _CH), jnp.float32),
        jax.ShapeDtypeStruct((_B * _MAXC,), jnp.int32),
    ],
    mesh=plsc.VectorSubcoreMesh(core_axis_name="c", subcore_axis_name="s"),
    compiler_params=pltpu.CompilerParams(needs_layout_passes=False),
    scratch_types=[
        pltpu.VMEM((_NCH,), jnp.float32),
        pltpu.VMEM((16,), jnp.float32),
        pltpu.VMEM((_MAXC,), jnp.int32),
        pltpu.VMEM((_MAXC,), jnp.int32),
        pltpu.VMEM((_MAXC, _CH), jnp.float32),
        pltpu.SemaphoreType.DMA,
    ],
)
def _sc_kernel(sc2, cmf, t32f, vout, lout, *scratch):
    _sc_gather(sc2, cmf, t32f, vout, lout, *scratch)


def _mask_kernel(x_ref, w_ref, b_ref, h_ref, vals_ref, cid_ref, t32_ref,
                 o_ref, tv_scr, ti_scr, wk_scr, tc_scr):
    i = pl.program_id(0)

    @pl.when(i == 0)
    def _():
        cid3 = cid_ref[:][:, :, None]
        t32r = t32_ref[:, 0:1][:, :, None]
        v = vals_ref[:]
        wk_scr[:] = jnp.where((cid3 < _NCH) & (v >= t32r), v,
                              -1.0).reshape(_B, _MAXC * _CH)
        ones = jnp.ones((_MAXC * _CH, 128), jnp.float32)

        def matcnt(ind):
            return lax.dot_general(
                ind.astype(jnp.float32), ones, (((1,), (0,)), ((), ())),
                preferred_element_type=jnp.float32)[:, 0:1]

        def body(k, p):
            cand = p | (1 << (30 - k))
            cf = lax.bitcast_convert_type(cand, jnp.float32)
            cnt = matcnt(wk_scr[:] >= cf)
            return jnp.where(cnt >= _K, cand, p)

        t_int = lax.fori_loop(0, 31, body, jnp.zeros((_B, 1), jnp.int32))
        t_f = lax.bitcast_convert_type(t_int, jnp.float32)
        n_gt = matcnt(wk_scr[:] > t_f)
        quota = _K - n_gt

        cols3 = cid3 * _CH + lax.broadcasted_iota(
            jnp.int32, (_B, _MAXC, _CH), 2)
        tc_scr[:] = jnp.where(wk_scr[:].reshape(_B, _MAXC, _CH) == t_f[:, :, None],
                              cols3, jnp.int32(1 << 20)).reshape(
                                  _B, _MAXC * _CH)

        def ibody(k, p):
            cand = p | (1 << (14 - k))
            g = matcnt(tc_scr[:] < cand)
            return jnp.where(g < quota, cand, p)

        idx_t = lax.fori_loop(0, 15, ibody, jnp.zeros((_B, 1), jnp.int32))
        tv_scr[:] = t_f
        ti_scr[:] = idx_t

    s = _scores_tile(x_ref, w_ref, b_ref, h_ref)
    t = tv_scr[:]
    it = ti_scr[:]
    col = i * _TN + lax.broadcasted_iota(jnp.int32, (_B, _TN), 1)
    keep = (s > t) | ((s == t) & (col <= it))
    o_ref[:] = jnp.where(keep, s, 0.0)


def kernel(x, W, b, health):
    b2 = b.reshape(1, _N)
    h2 = health.reshape(1, _N)
    scores, t32, cm = pl.pallas_call(
        _score_kernel,
        grid=(_NT,),
        in_specs=[
            pl.BlockSpec((_B, _D), lambda i: (0, 0)),
            pl.BlockSpec((_TN, _D), lambda i: (i, 0)),
            pl.BlockSpec((1, _TN), lambda i: (0, i)),
            pl.BlockSpec((1, _TN), lambda i: (0, i)),
        ],
        out_specs=[
            pl.BlockSpec((_B, _CPT, _CH), lambda i: (0, i, 0)),
            pl.BlockSpec((_B, 128), lambda i: (0, 0)),
            pl.BlockSpec((1, _CPT, _B), lambda i: (i, 0, 0)),
        ],
        out_shape=[
            jax.ShapeDtypeStruct((_B, _NCH, _CH), jnp.float32),
            jax.ShapeDtypeStruct((_B, 128), jnp.float32),
            jax.ShapeDtypeStruct((_NT, _CPT, _B), jnp.float32),
        ],
        scratch_shapes=[pltpu.VMEM((_NT, _CPT, _B), jnp.float32)],
    )(x, W, b2, h2)

    cmf = jnp.transpose(cm, (2, 0, 1)).reshape(_B * _NCH)
    vout, lout = _sc_kernel(
        scores.reshape(_B * _NCH, _CH),
        cmf,
        t32.reshape(_B * 128),
    )

    return pl.pallas_call(
        _mask_kernel,
        grid=(_NT,),
        in_specs=[
            pl.BlockSpec((_B, _D), lambda i: (0, 0)),
            pl.BlockSpec((_TN, _D), lambda i: (i, 0)),
            pl.BlockSpec((1, _TN), lambda i: (0, i)),
            pl.BlockSpec((1, _TN), lambda i: (0, i)),
            pl.BlockSpec((_B, _MAXC, _CH), lambda i: (0, 0, 0)),
            pl.BlockSpec((_B, _MAXC), lambda i: (0, 0)),
            pl.BlockSpec((_B, 128), lambda i: (0, 0)),
        ],
        out_specs=pl.BlockSpec((_B, _TN), lambda i: (0, i)),
        out_shape=jax.ShapeDtypeStruct((_B, _N), jnp.float32),
        scratch_shapes=[pltpu.VMEM((_B, 1), jnp.float32),
                        pltpu.VMEM((_B, 1), jnp.int32),
                        pltpu.VMEM((_B, _MAXC * _CH), jnp.float32),
                        pltpu.VMEM((_B, _MAXC * _CH), jnp.int32)],
    )(x, W, b2, h2, vout.reshape(_B, _MAXC, _CH), lout.reshape(_B, _MAXC),
      t32)
